# baseline (speedup 1.0000x reference)
import functools

import jax
import jax.numpy as jnp
from jax import lax
from jax.experimental import pallas as pl
from jax.experimental.pallas import tpu as pltpu
from jax.experimental.pallas import tpu_sc as plsc

_NC, _NS = 2, 16
_NW = _NC * _NS
_GATHER_CHUNK = 64


def _sc_gather(table, idx_flat):
    total, d = idx_flat.shape[0], table.shape[1]
    b_per_w = total // _NW
    n_chunks = b_per_w // _GATHER_CHUNK
    mesh = plsc.VectorSubcoreMesh(core_axis_name="c", subcore_axis_name="s")

    @functools.partial(
        pl.kernel,
        mesh=mesh,
        out_type=jax.ShapeDtypeStruct((total, d), table.dtype),
        scratch_types=[
            pltpu.VMEM((b_per_w,), jnp.int32),
            pltpu.VMEM((_GATHER_CHUNK, d), table.dtype),
            pltpu.SemaphoreType.DMA,
        ],
    )
    def gather_kernel(table_hbm, idx_hbm, out_hbm, idx_v, rows_v, sem):
        wid = lax.axis_index("s") * _NC + lax.axis_index("c")
        base = wid * b_per_w
        pltpu.sync_copy(idx_hbm.at[pl.ds(base, b_per_w)], idx_v)

        @pl.loop(0, n_chunks)
        def _(c):
            off = c * _GATHER_CHUNK
            pltpu.async_copy(
                table_hbm.at[idx_v.at[pl.ds(off, _GATHER_CHUNK)]], rows_v, sem
            ).wait()
            pltpu.sync_copy(rows_v, out_hbm.at[pl.ds(base + off, _GATHER_CHUNK)])

    return gather_kernel(table, idx_flat)


def _mlp_body(pos_blocks, bm, x_ref, p_ref, w_ref, b_ref, o_ref):
    i = pl.program_id(0)
    p = p_ref[pl.ds((i % pos_blocks) * bm, bm), :]
    h = (x_ref[...] + p).astype(jnp.bfloat16)
    acc = jnp.dot(h, w_ref[...], preferred_element_type=jnp.float32)
    acc = acc + b_ref[...]
    o_ref[...] = 0.5 * acc * (1.0 + lax.erf(acc * 0.7071067811865476))


def _tc_mlp(hidden, pos_emb, w_bf16, bias_2d, seq_len):
    m, k = hidden.shape
    n = w_bf16.shape[1]
    bm = 512
    pos_blocks = seq_len // bm
    grid = (m // bm,)

    return pl.pallas_call(
        functools.partial(_mlp_body, pos_blocks, bm),
        grid=grid,
        in_specs=[
            pl.BlockSpec((bm, k), lambda i: (i, 0)),
            pl.BlockSpec((seq_len, k), lambda i: (0, 0)),
            pl.BlockSpec((k, n), lambda i: (0, 0)),
            pl.BlockSpec((1, n), lambda i: (0, 0)),
        ],
        out_specs=pl.BlockSpec((bm, n), lambda i: (i, 0)),
        out_shape=jax.ShapeDtypeStruct((m, n), jnp.float32),
        compiler_params=pltpu.CompilerParams(
            dimension_semantics=("parallel",),
        ),
    )(hidden, pos_emb, w_bf16, bias_2d)


def kernel(tokens, token_emb, pos_emb, W, b):
    batch, seq = tokens.shape
    n = W.shape[1]
    idx = tokens.reshape(batch * seq).astype(jnp.int32)
    hidden = _sc_gather(token_emb, idx)
    out = _tc_mlp(hidden, pos_emb, W.astype(jnp.bfloat16), b.reshape(1, n), seq)
    return out.reshape(batch, seq, n)


# speedup vs baseline: 1.0172x; 1.0172x over previous
import functools

import jax
import jax.numpy as jnp
from jax import lax
from jax.experimental import pallas as pl
from jax.experimental.pallas import tpu as pltpu
from jax.experimental.pallas import tpu_sc as plsc

_NC, _NS = 2, 16
_NW = _NC * _NS
_GATHER_CHUNK = 64
_N_CHUNKS = 2
_BM = 512


def _sc_gather(table, idx_flat):
    total, d = idx_flat.shape[0], table.shape[1]
    b_per_w = total // _NW
    n_chunks = b_per_w // _GATHER_CHUNK
    mesh = plsc.VectorSubcoreMesh(core_axis_name="c", subcore_axis_name="s")

    @functools.partial(
        pl.kernel,
        mesh=mesh,
        out_type=jax.ShapeDtypeStruct((total, d), table.dtype),
        scratch_types=[
            pltpu.VMEM((b_per_w,), jnp.int32),
            pltpu.VMEM((_GATHER_CHUNK, d), table.dtype),
            pltpu.SemaphoreType.DMA,
        ],
    )
    def gather_kernel(table_hbm, idx_hbm, out_hbm, idx_v, rows_v, sem):
        wid = lax.axis_index("s") * _NC + lax.axis_index("c")
        base = wid * b_per_w
        pltpu.sync_copy(idx_hbm.at[pl.ds(base, b_per_w)], idx_v)

        @pl.loop(0, n_chunks)
        def _(c):
            off = c * _GATHER_CHUNK
            pltpu.async_copy(
                table_hbm.at[idx_v.at[pl.ds(off, _GATHER_CHUNK)]], rows_v, sem
            ).wait()
            pltpu.sync_copy(rows_v, out_hbm.at[pl.ds(base + off, _GATHER_CHUNK)])

    return gather_kernel(table, idx_flat)


def _mlp_body(x_ref, p_ref, w_ref, b_ref, *rest):
    o_ref = rest[-1]
    h = (x_ref[...] + p_ref[...]).astype(jnp.bfloat16)
    acc = jnp.dot(h, w_ref[...], preferred_element_type=jnp.float32)
    acc = acc + b_ref[...]
    o_ref[...] = 0.5 * acc * (1.0 + lax.erf(acc * 0.7071067811865476))


def _tc_mlp_chunk(hidden_chunk, pos_emb, w_bf16, bias_2d, seq_len,
                  m_total, row_base, out_prev):
    mc, k = hidden_chunk.shape
    n = w_bf16.shape[1]
    pos_blocks = seq_len // _BM
    base_blocks = row_base // _BM
    grid = (mc // _BM,)

    in_specs = [
        pl.BlockSpec((_BM, k), lambda i: (i, 0)),
        pl.BlockSpec((_BM, k), lambda i: (i % pos_blocks, 0)),
        pl.BlockSpec((k, n), lambda i: (0, 0)),
        pl.BlockSpec((1, n), lambda i: (0, 0)),
    ]
    args = [hidden_chunk, pos_emb, w_bf16, bias_2d]
    io_aliases = {}
    if out_prev is not None:
        in_specs.append(pl.BlockSpec(memory_space=pl.ANY))
        args.append(out_prev)
        io_aliases = {4: 0}

    return pl.pallas_call(
        _mlp_body,
        grid=grid,
        in_specs=in_specs,
        out_specs=pl.BlockSpec((_BM, n), lambda i: (base_blocks + i, 0)),
        out_shape=jax.ShapeDtypeStruct((m_total, n), jnp.float32),
        input_output_aliases=io_aliases,
        compiler_params=pltpu.CompilerParams(
            dimension_semantics=("arbitrary",),
        ),
    )(*args)


def kernel(tokens, token_emb, pos_emb, W, b):
    batch, seq = tokens.shape
    n = W.shape[1]
    m_total = batch * seq
    chunk = m_total // _N_CHUNKS
    idx = tokens.reshape(m_total).astype(jnp.int32)
    w_bf16 = W.astype(jnp.bfloat16)
    bias_2d = b.reshape(1, n)

    hiddens = [
        _sc_gather(token_emb, lax.slice(idx, (c * chunk,), ((c + 1) * chunk,)))
        for c in range(_N_CHUNKS)
    ]
    out = None
    for c in range(_N_CHUNKS):
        out = _tc_mlp_chunk(hiddens[c], pos_emb, w_bf16, bias_2d, seq,
                            m_total, c * chunk, out)
    return out.reshape(batch, seq, n)


# speedup vs baseline: 1.0244x; 1.0071x over previous
import functools

import jax
import jax.numpy as jnp
from jax import lax
from jax.experimental import pallas as pl
from jax.experimental.pallas import tpu as pltpu
from jax.experimental.pallas import tpu_sc as plsc

_NC, _NS = 2, 16
_NW = _NC * _NS
_GATHER_CHUNK = 32
_N_CHUNKS = 1
_BM = 512


def _sc_gather(table, idx_flat):
    total, d = idx_flat.shape[0], table.shape[1]
    b_per_w = total // _NW
    n_chunks = b_per_w // _GATHER_CHUNK
    mesh = plsc.VectorSubcoreMesh(core_axis_name="c", subcore_axis_name="s")

    @functools.partial(
        pl.kernel,
        mesh=mesh,
        out_type=jax.ShapeDtypeStruct((total, d), table.dtype),
        scratch_types=[
            pltpu.VMEM((b_per_w,), jnp.int32),
            pltpu.VMEM((_GATHER_CHUNK, d), table.dtype),
            pltpu.VMEM((_GATHER_CHUNK, d), table.dtype),
            pltpu.SemaphoreType.DMA,
            pltpu.SemaphoreType.DMA,
            pltpu.SemaphoreType.DMA,
            pltpu.SemaphoreType.DMA,
        ],
    )
    def gather_kernel(table_hbm, idx_hbm, out_hbm, idx_v, rows0, rows1,
                      gsem0, gsem1, wsem0, wsem1):
        wid = lax.axis_index("s") * _NC + lax.axis_index("c")
        base = wid * b_per_w
        bufs = (rows0, rows1)
        gsems = (gsem0, gsem1)
        wsems = (wsem0, wsem1)
        pltpu.sync_copy(idx_hbm.at[pl.ds(base, b_per_w)], idx_v)

        def gather(c):
            return pltpu.make_async_copy(
                table_hbm.at[idx_v.at[pl.ds(c * _GATHER_CHUNK, _GATHER_CHUNK)]],
                bufs[c % 2], gsems[c % 2])

        def writeback(c):
            return pltpu.make_async_copy(
                bufs[c % 2],
                out_hbm.at[pl.ds(base + c * _GATHER_CHUNK, _GATHER_CHUNK)],
                wsems[c % 2])

        gather(0).start()
        if n_chunks > 1:
            gather(1).start()
        for c in range(n_chunks):
            gather(c).wait()
            writeback(c).start()
            writeback(c).wait()
            if c + 2 < n_chunks:
                gather(c + 2).start()

    return gather_kernel(table, idx_flat)


def _mlp_body(x_ref, p_ref, w_ref, b_ref, *rest):
    o_ref = rest[-1]
    h = (x_ref[...] + p_ref[...]).astype(jnp.bfloat16)
    acc = jnp.dot(h, w_ref[...], preferred_element_type=jnp.float32)
    acc = acc + b_ref[...]
    o_ref[...] = 0.5 * acc * (1.0 + lax.erf(acc * 0.7071067811865476))


def _tc_mlp_chunk(hidden_chunk, pos_emb, w_bf16, bias_2d, seq_len,
                  m_total, row_base, out_prev):
    mc, k = hidden_chunk.shape
    n = w_bf16.shape[1]
    pos_blocks = seq_len // _BM
    base_blocks = row_base // _BM
    grid = (mc // _BM,)

    in_specs = [
        pl.BlockSpec((_BM, k), lambda i: (i, 0)),
        pl.BlockSpec((_BM, k), lambda i: (i % pos_blocks, 0)),
        pl.BlockSpec((k, n), lambda i: (0, 0)),
        pl.BlockSpec((1, n), lambda i: (0, 0)),
    ]
    args = [hidden_chunk, pos_emb, w_bf16, bias_2d]
    io_aliases = {}
    if out_prev is not None:
        in_specs.append(pl.BlockSpec(memory_space=pl.ANY))
        args.append(out_prev)
        io_aliases = {4: 0}

    return pl.pallas_call(
        _mlp_body,
        grid=grid,
        in_specs=in_specs,
        out_specs=pl.BlockSpec((_BM, n), lambda i: (base_blocks + i, 0)),
        out_shape=jax.ShapeDtypeStruct((m_total, n), jnp.float32),
        input_output_aliases=io_aliases,
        compiler_params=pltpu.CompilerParams(
            dimension_semantics=("arbitrary",),
        ),
    )(*args)


def kernel(tokens, token_emb, pos_emb, W, b):
    batch, seq = tokens.shape
    n = W.shape[1]
    m_total = batch * seq
    chunk = m_total // _N_CHUNKS
    idx = tokens.reshape(m_total).astype(jnp.int32)
    w_bf16 = W.astype(jnp.bfloat16)
    bias_2d = b.reshape(1, n)

    hiddens = [
        _sc_gather(token_emb, lax.slice(idx, (c * chunk,), ((c + 1) * chunk,)))
        for c in range(_N_CHUNKS)
    ]
    out = None
    for c in range(_N_CHUNKS):
        out = _tc_mlp_chunk(hiddens[c], pos_emb, w_bf16, bias_2d, seq,
                            m_total, c * chunk, out)
    return out.reshape(batch, seq, n)


# speedup vs baseline: 1.0323x; 1.0077x over previous
import functools

import jax
import jax.numpy as jnp
from jax import lax
from jax.experimental import pallas as pl
from jax.experimental.pallas import tpu as pltpu
from jax.experimental.pallas import tpu_sc as plsc

_NC, _NS = 2, 16
_NW = _NC * _NS
_GATHER_CHUNK = 32
_N_CHUNKS = 1
_BM = 512


def _sc_gather(table, idx_flat):
    total, d = idx_flat.shape[0], table.shape[1]
    b_per_w = total // _NW
    n_chunks = b_per_w // _GATHER_CHUNK
    mesh = plsc.VectorSubcoreMesh(core_axis_name="c", subcore_axis_name="s")

    @functools.partial(
        pl.kernel,
        mesh=mesh,
        out_type=jax.ShapeDtypeStruct((total, d), table.dtype),
        scratch_types=[
            pltpu.VMEM((b_per_w,), jnp.int32),
            pltpu.VMEM((_GATHER_CHUNK, d), table.dtype),
            pltpu.VMEM((_GATHER_CHUNK, d), table.dtype),
            pltpu.SemaphoreType.DMA,
            pltpu.SemaphoreType.DMA,
            pltpu.SemaphoreType.DMA,
            pltpu.SemaphoreType.DMA,
        ],
    )
    def gather_kernel(table_hbm, idx_hbm, out_hbm, idx_v, rows0, rows1,
                      gsem0, gsem1, wsem0, wsem1):
        wid = lax.axis_index("s") * _NC + lax.axis_index("c")
        base = wid * b_per_w
        bufs = (rows0, rows1)
        gsems = (gsem0, gsem1)
        wsems = (wsem0, wsem1)
        pltpu.sync_copy(idx_hbm.at[pl.ds(base, b_per_w)], idx_v)

        def gather(c):
            return pltpu.make_async_copy(
                table_hbm.at[idx_v.at[pl.ds(c * _GATHER_CHUNK, _GATHER_CHUNK)]],
                bufs[c % 2], gsems[c % 2])

        def writeback(c):
            return pltpu.make_async_copy(
                bufs[c % 2],
                out_hbm.at[pl.ds(base + c * _GATHER_CHUNK, _GATHER_CHUNK)],
                wsems[c % 2])

        gather(0).start()
        if n_chunks > 1:
            gather(1).start()
        for c in range(n_chunks):
            gather(c).wait()
            writeback(c).start()
            writeback(c).wait()
            if c + 2 < n_chunks:
                gather(c + 2).start()

    return gather_kernel(table, idx_flat)


def _mlp_body(x_ref, p_ref, w_ref, b_ref, *rest):
    o_ref = rest[-1]
    h = x_ref[...] + p_ref[...]
    acc = jnp.dot(h, w_ref[...], precision=lax.Precision.DEFAULT,
                  preferred_element_type=jnp.float32)
    acc = acc + b_ref[...]
    o_ref[...] = 0.5 * acc * (1.0 + lax.erf(acc * 0.7071067811865476))


def _tc_mlp_chunk(hidden_chunk, pos_emb, w_bf16, bias_2d, seq_len,
                  m_total, row_base, out_prev):
    mc, k = hidden_chunk.shape
    n = w_bf16.shape[1]
    pos_blocks = seq_len // _BM
    base_blocks = row_base // _BM
    grid = (mc // _BM,)

    in_specs = [
        pl.BlockSpec((_BM, k), lambda i: (i, 0)),
        pl.BlockSpec((_BM, k), lambda i: (i % pos_blocks, 0)),
        pl.BlockSpec((k, n), lambda i: (0, 0)),
        pl.BlockSpec((1, n), lambda i: (0, 0)),
    ]
    args = [hidden_chunk, pos_emb, w_bf16, bias_2d]
    io_aliases = {}
    if out_prev is not None:
        in_specs.append(pl.BlockSpec(memory_space=pl.ANY))
        args.append(out_prev)
        io_aliases = {4: 0}

    return pl.pallas_call(
        _mlp_body,
        grid=grid,
        in_specs=in_specs,
        out_specs=pl.BlockSpec((_BM, n), lambda i: (base_blocks + i, 0)),
        out_shape=jax.ShapeDtypeStruct((m_total, n), jnp.float32),
        input_output_aliases=io_aliases,
        compiler_params=pltpu.CompilerParams(
            dimension_semantics=("arbitrary",),
        ),
    )(*args)


def kernel(tokens, token_emb, pos_emb, W, b):
    batch, seq = tokens.shape
    n = W.shape[1]
    m_total = batch * seq
    chunk = m_total // _N_CHUNKS
    idx = tokens.reshape(m_total).astype(jnp.int32)
    bias_2d = b.reshape(1, n)

    hiddens = [
        _sc_gather(token_emb, lax.slice(idx, (c * chunk,), ((c + 1) * chunk,)))
        for c in range(_N_CHUNKS)
    ]
    out = None
    for c in range(_N_CHUNKS):
        out = _tc_mlp_chunk(hiddens[c], pos_emb, W, bias_2d, seq,
                            m_total, c * chunk, out)
    return out.reshape(batch, seq, n)
